# Initial kernel scaffold; baseline (speedup 1.0000x reference)
import jax
import jax.numpy as jnp
from jax.experimental import pallas as pl


def kernel(x, edge_index, edge_attr, W, att_src, att_dst, lin_edge_W, att_edge, bias):
    raise NotImplementedError("write your pallas kernel here")


# speedup vs baseline: 18.3249x; 18.3249x over previous
import functools

import jax
import jax.numpy as jnp
from jax import lax
from jax.experimental import pallas as pl
from jax.experimental.pallas import tpu as pltpu
from jax.experimental.pallas import tpu_sc as plsc

N = 10000
E = 320000
D = 128
DE = 16

NC = 2
NS = 16
NW = NC * NS
LANE = 16

NPAD = 10240
ECH = 79
ET = ECH * 128
EPAD = NW * ET
PAD_E = EPAD - E

EB = EPAD // 16
NB = NPAD // 16


def _k0_body(x_r, ea_r, w_r, asw_r, adw_r, lew_r, aew_r,
             h_r, asrc_r, adst_r, ae_r):
    xb = x_r[...]
    h = lax.dot_general(xb, w_r[...], (((1,), (1,)), ((), ())),
                        preferred_element_type=jnp.float32)
    h_r[...] = h
    asrc_r[...] = jnp.sum(h * asw_r[...], axis=1)[None, None, :]
    adst_r[...] = jnp.sum(h * adw_r[...], axis=1)[None, None, :]
    v = lax.dot_general(aew_r[...], lew_r[...], (((1,), (0,)), ((), ())),
                        preferred_element_type=jnp.float32)
    ae_r[...] = jnp.sum(ea_r[...] * v, axis=1)[None, None, :]


def _k0(xp, eap, W, att_src, att_dst, lin_edge_W, att_edge):
    return pl.pallas_call(
        _k0_body,
        grid=(16,),
        in_specs=[
            pl.BlockSpec((NB, D), lambda i: (i, 0)),
            pl.BlockSpec((EB, DE), lambda i: (i, 0)),
            pl.BlockSpec((D, D), lambda i: (0, 0)),
            pl.BlockSpec((1, D), lambda i: (0, 0)),
            pl.BlockSpec((1, D), lambda i: (0, 0)),
            pl.BlockSpec((D, DE), lambda i: (0, 0)),
            pl.BlockSpec((1, D), lambda i: (0, 0)),
        ],
        out_specs=[
            pl.BlockSpec((NB, D), lambda i: (i, 0)),
            pl.BlockSpec((1, 1, NB), lambda i: (i, 0, 0)),
            pl.BlockSpec((1, 1, NB), lambda i: (i, 0, 0)),
            pl.BlockSpec((1, 1, EB), lambda i: (i, 0, 0)),
        ],
        out_shape=[
            jax.ShapeDtypeStruct((NPAD, D), jnp.float32),
            jax.ShapeDtypeStruct((16, 1, NB), jnp.float32),
            jax.ShapeDtypeStruct((16, 1, NB), jnp.float32),
            jax.ShapeDtypeStruct((16, 1, EB), jnp.float32),
        ],
    )(xp, eap, W, att_src, att_dst, lin_edge_W, att_edge)


def _k1_body(srcp, dstp, aep, asrc, adst, zeros1,
             alpha_o, degp_o, saep_o, tmax_o,
             src_v, dst_v, ae_v, alpha_v, asrc_v, adst_v, ones_v, tmax_v,
             deg_sh, sae_sh):
    cid = lax.axis_index("c")
    sid = lax.axis_index("s")
    wid = sid * NC + cid

    pltpu.sync_copy(srcp.at[wid], src_v)
    pltpu.sync_copy(dstp.at[wid], dst_v)
    pltpu.sync_copy(aep.at[wid], ae_v)
    pltpu.sync_copy(asrc, asrc_v)
    pltpu.sync_copy(adst, adst_v)

    for g in range(8):
        ones_v[pl.ds(g * LANE, LANE)] = jnp.full((LANE,), 1.0, jnp.float32)

    pltpu.sync_copy(zeros1, deg_sh.at[pl.ds(sid * 640, 640)])
    pltpu.sync_copy(zeros1, sae_sh.at[pl.ds(sid * 640, 640)])
    plsc.subcore_barrier()

    def chunk(j, m):
        for g in range(8):
            off = pl.ds(g * LANE, LANE)
            s16 = src_v[j, off]
            d16 = dst_v[j, off]
            ae16 = ae_v[j, off]
            al = plsc.load_gather(asrc_v, [s16]) \
                + plsc.load_gather(adst_v, [d16]) + ae16
            al = jnp.maximum(al, al * 0.2)
            alpha_v[j, off] = al
            m = jnp.maximum(m, al)
        pltpu.sync_copy(ones_v, deg_sh.at[dst_v.at[j]], add=True)
        pltpu.sync_copy(ae_v.at[j], sae_sh.at[dst_v.at[j]], add=True)
        return m

    m0 = jnp.full((LANE,), -1e30, jnp.float32)
    m = lax.fori_loop(0, ECH, chunk, m0)
    tmax_v[...] = m

    pltpu.sync_copy(alpha_v, alpha_o.at[wid])
    pltpu.sync_copy(tmax_v, tmax_o.at[wid])
    plsc.subcore_barrier()
    base = cid * NPAD + sid * 640
    pltpu.sync_copy(deg_sh.at[pl.ds(sid * 640, 640)],
                    degp_o.at[pl.ds(base, 640)])
    pltpu.sync_copy(sae_sh.at[pl.ds(sid * 640, 640)],
                    saep_o.at[pl.ds(base, 640)])


def _k1(srcp, dstp, aep, asrc, adst, zeros1):
    mesh = plsc.VectorSubcoreMesh(core_axis_name="c", subcore_axis_name="s",
                                  num_cores=NC, num_subcores=NS)
    f = pl.kernel(
        _k1_body,
        out_type=[
            jax.ShapeDtypeStruct((NW, ECH, 128), jnp.float32),
            jax.ShapeDtypeStruct((NC * NPAD,), jnp.float32),
            jax.ShapeDtypeStruct((NC * NPAD,), jnp.float32),
            jax.ShapeDtypeStruct((NW, LANE), jnp.float32),
        ],
        mesh=mesh,
        compiler_params=pltpu.CompilerParams(needs_layout_passes=False),
        scratch_types=[
            pltpu.VMEM((ECH, 128), jnp.int32),
            pltpu.VMEM((ECH, 128), jnp.int32),
            pltpu.VMEM((ECH, 128), jnp.float32),
            pltpu.VMEM((ECH, 128), jnp.float32),
            pltpu.VMEM((NPAD,), jnp.float32),
            pltpu.VMEM((NPAD,), jnp.float32),
            pltpu.VMEM((128,), jnp.float32),
            pltpu.VMEM((LANE,), jnp.float32),
            pltpu.VMEM_SHARED((NPAD,), jnp.float32),
            pltpu.VMEM_SHARED((NPAD,), jnp.float32),
        ],
    )
    return f(srcp, dstp, aep, asrc, adst, zeros1)


def _k2_body(degp_r, saep_r, asrc_r, adst_r, tmax_r, aloop_o, gmax_o):
    deg = jnp.sum(degp_r[...], axis=0)
    sae = jnp.sum(saep_r[...], axis=0)
    ae_loop = sae / jnp.maximum(deg, 1.0)
    al = asrc_r[...] + adst_r[...] + ae_loop
    al = jnp.maximum(al, al * 0.2)
    aloop_o[...] = al
    gmax = jnp.maximum(jnp.max(tmax_r[...]), jnp.max(al))
    gmax_o[...] = jnp.full((8, 128), gmax, jnp.float32)


def _k2(degp, saep, asrc, adst, tmax):
    return pl.pallas_call(
        _k2_body,
        out_shape=[
            jax.ShapeDtypeStruct((NPAD,), jnp.float32),
            jax.ShapeDtypeStruct((8, 128), jnp.float32),
        ],
    )(degp.reshape(NC, NPAD), saep.reshape(NC, NPAD), asrc, adst, tmax)


def _k3_body(srcp, dstp, alphap, gmaxp, h, zeros1, zeros2,
             denomp_o, acc_o,
             src_v, dst_v, expv, rows_v, gmax_v,
             denom_sh, acc_sh, sem):
    cid = lax.axis_index("c")
    sid = lax.axis_index("s")
    wid = sid * NC + cid

    pltpu.sync_copy(srcp.at[wid], src_v)
    pltpu.sync_copy(dstp.at[wid], dst_v)
    pltpu.sync_copy(alphap.at[wid], expv)
    pltpu.sync_copy(gmaxp.at[0], gmax_v)

    pltpu.sync_copy(zeros1, denom_sh.at[pl.ds(sid * 640, 640)])
    pltpu.sync_copy(zeros2, acc_sh.at[pl.ds(sid * 640, 640)])
    plsc.subcore_barrier()

    gsp = gmax_v[pl.ds(0, LANE)]

    def chunk(j, carry):
        for g in range(8):
            off = pl.ds(g * LANE, LANE)
            expv[j, off] = jnp.exp(expv[j, off] - gsp)
        pltpu.sync_copy(expv.at[j], denom_sh.at[dst_v.at[j]], add=True)
        pltpu.async_copy(h.at[src_v.at[j]], rows_v, sem).wait()

        def scale(r, c):
            spl = plsc.load_gather(
                expv, [jnp.full((LANE,), j, jnp.int32),
                       jnp.full((LANE,), r, jnp.int32)])
            for g in range(8):
                off = pl.ds(g * LANE, LANE)
                rows_v[r, off] = rows_v[r, off] * spl
            return c
        lax.fori_loop(0, 128, scale, 0)
        pltpu.sync_copy(rows_v, acc_sh.at[dst_v.at[j]], add=True)
        return carry

    lax.fori_loop(0, ECH, chunk, 0)

    plsc.subcore_barrier()
    base = cid * NPAD + sid * 640
    pltpu.sync_copy(denom_sh.at[pl.ds(sid * 640, 640)],
                    denomp_o.at[pl.ds(base, 640)])
    pltpu.sync_copy(acc_sh.at[pl.ds(sid * 640, 640)],
                    acc_o.at[pl.ds(base, 640)])


def _k3(srcp, dstp, alphap, gmaxp, h, zeros1, zeros2):
    mesh = plsc.VectorSubcoreMesh(core_axis_name="c", subcore_axis_name="s",
                                  num_cores=NC, num_subcores=NS)
    f = pl.kernel(
        _k3_body,
        out_type=[
            jax.ShapeDtypeStruct((NC * NPAD,), jnp.float32),
            jax.ShapeDtypeStruct((NC * NPAD, D), jnp.float32),
        ],
        mesh=mesh,
        compiler_params=pltpu.CompilerParams(needs_layout_passes=False),
        scratch_types=[
            pltpu.VMEM((ECH, 128), jnp.int32),
            pltpu.VMEM((ECH, 128), jnp.int32),
            pltpu.VMEM((ECH, 128), jnp.float32),
            pltpu.VMEM((128, D), jnp.float32),
            pltpu.VMEM((128,), jnp.float32),
            pltpu.VMEM_SHARED((NPAD,), jnp.float32),
            pltpu.VMEM_SHARED((NPAD, D), jnp.float32),
            pltpu.SemaphoreType.DMA,
        ],
    )
    return f(srcp, dstp, alphap, gmaxp, h, zeros1, zeros2)


def _k4_body(acc0_r, acc1_r, den0_r, den1_r, aloop_r, gmax_r, h_r, bias_r,
             out_r):
    gm = jnp.max(gmax_r[...])
    el = jnp.exp(aloop_r[...] - gm)
    den = den0_r[...] + den1_r[...] + el + 1e-16
    num = acc0_r[...] + acc1_r[...] + el * h_r[...]
    out = num / den + bias_r[...]
    out_r[...] = jnp.maximum(out, 0.0)


def _k4(acc, denomp, aloop, gmaxp, h, bias):
    acc2 = acc.reshape(NC, NPAD, D)
    den2 = denomp.reshape(NC, NPAD, 1)
    nb = NPAD // 8
    out = pl.pallas_call(
        _k4_body,
        grid=(8,),
        in_specs=[
            pl.BlockSpec((nb, D), lambda i: (i, 0)),
            pl.BlockSpec((nb, D), lambda i: (i, 0)),
            pl.BlockSpec((nb, 1), lambda i: (i, 0)),
            pl.BlockSpec((nb, 1), lambda i: (i, 0)),
            pl.BlockSpec((nb, 1), lambda i: (i, 0)),
            pl.BlockSpec((8, 128), lambda i: (0, 0)),
            pl.BlockSpec((nb, D), lambda i: (i, 0)),
            pl.BlockSpec((1, D), lambda i: (0, 0)),
        ],
        out_specs=pl.BlockSpec((nb, D), lambda i: (i, 0)),
        out_shape=jax.ShapeDtypeStruct((NPAD, D), jnp.float32),
    )(acc2[0], acc2[1], den2[0], den2[1], aloop.reshape(NPAD, 1), gmaxp, h,
      bias)
    return out[:N]


@jax.jit
def kernel(x, edge_index, edge_attr, W, att_src, att_dst, lin_edge_W,
           att_edge, bias):
    src = edge_index[0]
    dst = edge_index[1]
    pad_nodes = (N + (jnp.arange(PAD_E, dtype=jnp.int32) % (NPAD - N)))
    srcp = jnp.concatenate([src, pad_nodes]).reshape(NW, ECH, 128)
    dstp = jnp.concatenate([dst, pad_nodes]).reshape(NW, ECH, 128)
    eap = jnp.concatenate(
        [edge_attr, jnp.zeros((PAD_E, DE), jnp.float32)], axis=0)
    xp = jnp.concatenate([x, jnp.zeros((NPAD - N, D), jnp.float32)], axis=0)
    zeros1 = jnp.zeros((640,), jnp.float32)
    zeros2 = jnp.zeros((640, D), jnp.float32)

    h, asrc, adst, ae = _k0(xp, eap, W, att_src.reshape(1, D),
                            att_dst.reshape(1, D), lin_edge_W,
                            att_edge.reshape(1, D))
    asrc = asrc.reshape(NPAD)
    adst = adst.reshape(NPAD)
    aep = ae.reshape(NW, ECH, 128)
    alpha, degp, saep, tmax = _k1(srcp, dstp, aep, asrc, adst, zeros1)
    aloop, gmaxp = _k2(degp, saep, asrc, adst, tmax)
    denomp, acc = _k3(srcp, dstp, alpha, gmaxp, h, zeros1, zeros2)
    return _k4(acc, denomp, aloop, gmaxp, h, bias.reshape(1, D))


# speedup vs baseline: 22.5693x; 1.2316x over previous
import functools

import jax
import jax.numpy as jnp
from jax import lax
from jax.experimental import pallas as pl
from jax.experimental.pallas import tpu as pltpu
from jax.experimental.pallas import tpu_sc as plsc

N = 10000
E = 320000
D = 128
DE = 16

NC = 2
NS = 16
NW = NC * NS
LANE = 16

NPAD = 10240
ECH = 79
ET = ECH * 128
EPAD = NW * ET
PAD_E = EPAD - E

ER = E // 8
EB = ER // 8
NB = NPAD // 8


def _k0_body(x_r, ea_r, w_r, asw_r, adw_r, lew_r, aew_r, emat_r, mask8_r,
             h_r, asrc_r, adst_r, ae_r):
    xb = x_r[...]
    h = lax.dot_general(xb, w_r[...], (((1,), (1,)), ((), ())),
                        preferred_element_type=jnp.float32)
    h_r[...] = h
    asrc_r[...] = jnp.sum(h * asw_r[...], axis=1)[None, None, :]
    adst_r[...] = jnp.sum(h * adw_r[...], axis=1)[None, None, :]
    v = lax.dot_general(aew_r[...], lew_r[...], (((1,), (0,)), ((), ())),
                        preferred_element_type=jnp.float32)
    vcol = lax.dot_general(emat_r[...], v, (((1,), (1,)), ((), ())),
                           preferred_element_type=jnp.float32)
    bmat = mask8_r[...] * vcol
    ae_r[...] = lax.dot_general(ea_r[...], bmat, (((1,), (0,)), ((), ())),
                                preferred_element_type=jnp.float32)


def _k0(xp, ea2, W, att_src, att_dst, lin_edge_W, att_edge, emat, mask8):
    return pl.pallas_call(
        _k0_body,
        grid=(8,),
        in_specs=[
            pl.BlockSpec((NB, D), lambda i: (i, 0)),
            pl.BlockSpec((EB, 128), lambda i: (i, 0)),
            pl.BlockSpec((D, D), lambda i: (0, 0)),
            pl.BlockSpec((1, D), lambda i: (0, 0)),
            pl.BlockSpec((1, D), lambda i: (0, 0)),
            pl.BlockSpec((D, DE), lambda i: (0, 0)),
            pl.BlockSpec((1, D), lambda i: (0, 0)),
            pl.BlockSpec((128, DE), lambda i: (0, 0)),
            pl.BlockSpec((128, 8), lambda i: (0, 0)),
        ],
        out_specs=[
            pl.BlockSpec((NB, D), lambda i: (i, 0)),
            pl.BlockSpec((1, 1, NB), lambda i: (i, 0, 0)),
            pl.BlockSpec((1, 1, NB), lambda i: (i, 0, 0)),
            pl.BlockSpec((EB, 8), lambda i: (i, 0)),
        ],
        out_shape=[
            jax.ShapeDtypeStruct((NPAD, D), jnp.float32),
            jax.ShapeDtypeStruct((8, 1, NB), jnp.float32),
            jax.ShapeDtypeStruct((8, 1, NB), jnp.float32),
            jax.ShapeDtypeStruct((ER, 8), jnp.float32),
        ],
    )(xp, ea2, W, att_src, att_dst, lin_edge_W, att_edge, emat, mask8)


def _k1_body(srcp, dstp, aep, asrc, adst, zeros1,
             alpha_o, degp_o, saep_o, tmax_o,
             src_v, dst_v, ae_v, alpha_v, asrc_v, adst_v, ones_v, tmax_v,
             deg_sh, sae_sh):
    cid = lax.axis_index("c")
    sid = lax.axis_index("s")
    wid = sid * NC + cid

    pltpu.sync_copy(srcp.at[wid], src_v)
    pltpu.sync_copy(dstp.at[wid], dst_v)
    pltpu.sync_copy(aep.at[wid], ae_v)
    pltpu.sync_copy(asrc, asrc_v)
    pltpu.sync_copy(adst, adst_v)

    for g in range(8):
        ones_v[pl.ds(g * LANE, LANE)] = jnp.full((LANE,), 1.0, jnp.float32)

    pltpu.sync_copy(zeros1, deg_sh.at[pl.ds(sid * 640, 640)])
    pltpu.sync_copy(zeros1, sae_sh.at[pl.ds(sid * 640, 640)])
    plsc.subcore_barrier()

    def chunk(j, m):
        for g in range(8):
            off = pl.ds(g * LANE, LANE)
            s16 = src_v[j, off]
            d16 = dst_v[j, off]
            ae16 = ae_v[j, off]
            al = plsc.load_gather(asrc_v, [s16]) \
                + plsc.load_gather(adst_v, [d16]) + ae16
            al = jnp.maximum(al, al * 0.2)
            alpha_v[j, off] = al
            m = jnp.maximum(m, al)
        pltpu.sync_copy(ones_v, deg_sh.at[dst_v.at[j]], add=True)
        pltpu.sync_copy(ae_v.at[j], sae_sh.at[dst_v.at[j]], add=True)
        return m

    m0 = jnp.full((LANE,), -1e30, jnp.float32)
    m = lax.fori_loop(0, ECH, chunk, m0)
    tmax_v[...] = m

    pltpu.sync_copy(alpha_v, alpha_o.at[wid])
    pltpu.sync_copy(tmax_v, tmax_o.at[wid])
    plsc.subcore_barrier()
    base = cid * NPAD + sid * 640
    pltpu.sync_copy(deg_sh.at[pl.ds(sid * 640, 640)],
                    degp_o.at[pl.ds(base, 640)])
    pltpu.sync_copy(sae_sh.at[pl.ds(sid * 640, 640)],
                    saep_o.at[pl.ds(base, 640)])


def _k1(srcp, dstp, aep, asrc, adst, zeros1):
    mesh = plsc.VectorSubcoreMesh(core_axis_name="c", subcore_axis_name="s",
                                  num_cores=NC, num_subcores=NS)
    f = pl.kernel(
        _k1_body,
        out_type=[
            jax.ShapeDtypeStruct((NW, ECH, 128), jnp.float32),
            jax.ShapeDtypeStruct((NC * NPAD,), jnp.float32),
            jax.ShapeDtypeStruct((NC * NPAD,), jnp.float32),
            jax.ShapeDtypeStruct((NW, LANE), jnp.float32),
        ],
        mesh=mesh,
        compiler_params=pltpu.CompilerParams(needs_layout_passes=False),
        scratch_types=[
            pltpu.VMEM((ECH, 128), jnp.int32),
            pltpu.VMEM((ECH, 128), jnp.int32),
            pltpu.VMEM((ECH, 128), jnp.float32),
            pltpu.VMEM((ECH, 128), jnp.float32),
            pltpu.VMEM((NPAD,), jnp.float32),
            pltpu.VMEM((NPAD,), jnp.float32),
            pltpu.VMEM((128,), jnp.float32),
            pltpu.VMEM((LANE,), jnp.float32),
            pltpu.VMEM_SHARED((NPAD,), jnp.float32),
            pltpu.VMEM_SHARED((NPAD,), jnp.float32),
        ],
    )
    return f(srcp, dstp, aep, asrc, adst, zeros1)


def _k2_body(degp_r, saep_r, asrc_r, adst_r, tmax_r, aloop_o, gmax_o):
    deg = jnp.sum(degp_r[...], axis=0)
    sae = jnp.sum(saep_r[...], axis=0)
    ae_loop = sae / jnp.maximum(deg, 1.0)
    al = asrc_r[...] + adst_r[...] + ae_loop
    al = jnp.maximum(al, al * 0.2)
    aloop_o[...] = al
    gmax = jnp.maximum(jnp.max(tmax_r[...]), jnp.max(al))
    gmax_o[...] = jnp.full((8, 128), gmax, jnp.float32)


def _k2(degp, saep, asrc, adst, tmax):
    return pl.pallas_call(
        _k2_body,
        out_shape=[
            jax.ShapeDtypeStruct((NPAD,), jnp.float32),
            jax.ShapeDtypeStruct((8, 128), jnp.float32),
        ],
    )(degp.reshape(NC, NPAD), saep.reshape(NC, NPAD), asrc, adst, tmax)


def _k3_body(srcp, dstp, alphap, gmaxp, h, zeros1, zeros2,
             denomp_o, acc_o,
             src_v, dst_v, expv, rows_v, gmax_v,
             denom_sh, acc_sh, sem):
    cid = lax.axis_index("c")
    sid = lax.axis_index("s")
    wid = sid * NC + cid

    pltpu.sync_copy(srcp.at[wid], src_v)
    pltpu.sync_copy(dstp.at[wid], dst_v)
    pltpu.sync_copy(alphap.at[wid], expv)
    pltpu.sync_copy(gmaxp.at[0], gmax_v)

    pltpu.sync_copy(zeros1, denom_sh.at[pl.ds(sid * 640, 640)])
    pltpu.sync_copy(zeros2, acc_sh.at[pl.ds(sid * 640, 640)])
    plsc.subcore_barrier()

    gsp = gmax_v[pl.ds(0, LANE)]

    def chunk(j, carry):
        for g in range(8):
            off = pl.ds(g * LANE, LANE)
            expv[j, off] = jnp.exp(expv[j, off] - gsp)
        pltpu.sync_copy(expv.at[j], denom_sh.at[dst_v.at[j]], add=True)
        pltpu.async_copy(h.at[src_v.at[j]], rows_v, sem).wait()

        def scale(r, c):
            spl = plsc.load_gather(
                expv, [jnp.full((LANE,), j, jnp.int32),
                       jnp.full((LANE,), r, jnp.int32)])
            for g in range(8):
                off = pl.ds(g * LANE, LANE)
                rows_v[r, off] = rows_v[r, off] * spl
            return c
        lax.fori_loop(0, 128, scale, 0)
        pltpu.sync_copy(rows_v, acc_sh.at[dst_v.at[j]], add=True)
        return carry

    lax.fori_loop(0, ECH, chunk, 0)

    plsc.subcore_barrier()
    base = cid * NPAD + sid * 640
    pltpu.sync_copy(denom_sh.at[pl.ds(sid * 640, 640)],
                    denomp_o.at[pl.ds(base, 640)])
    pltpu.sync_copy(acc_sh.at[pl.ds(sid * 640, 640)],
                    acc_o.at[pl.ds(base, 640)])


def _k3(srcp, dstp, alphap, gmaxp, h, zeros1, zeros2):
    mesh = plsc.VectorSubcoreMesh(core_axis_name="c", subcore_axis_name="s",
                                  num_cores=NC, num_subcores=NS)
    f = pl.kernel(
        _k3_body,
        out_type=[
            jax.ShapeDtypeStruct((NC * NPAD,), jnp.float32),
            jax.ShapeDtypeStruct((NC * NPAD, D), jnp.float32),
        ],
        mesh=mesh,
        compiler_params=pltpu.CompilerParams(needs_layout_passes=False),
        scratch_types=[
            pltpu.VMEM((ECH, 128), jnp.int32),
            pltpu.VMEM((ECH, 128), jnp.int32),
            pltpu.VMEM((ECH, 128), jnp.float32),
            pltpu.VMEM((128, D), jnp.float32),
            pltpu.VMEM((128,), jnp.float32),
            pltpu.VMEM_SHARED((NPAD,), jnp.float32),
            pltpu.VMEM_SHARED((NPAD, D), jnp.float32),
            pltpu.SemaphoreType.DMA,
        ],
    )
    return f(srcp, dstp, alphap, gmaxp, h, zeros1, zeros2)


def _k4_body(acc0_r, acc1_r, den0_r, den1_r, aloop_r, gmax_r, h_r, bias_r,
             out_r):
    gm = jnp.max(gmax_r[...])
    el = jnp.exp(aloop_r[...] - gm)
    den = den0_r[...] + den1_r[...] + el + 1e-16
    num = acc0_r[...] + acc1_r[...] + el * h_r[...]
    out = num / den + bias_r[...]
    out_r[...] = jnp.maximum(out, 0.0)


def _k4(acc, denomp, aloop, gmaxp, h, bias):
    acc2 = acc.reshape(NC, NPAD, D)
    den2 = denomp.reshape(NC, NPAD, 1)
    nb = NPAD // 8
    out = pl.pallas_call(
        _k4_body,
        grid=(8,),
        in_specs=[
            pl.BlockSpec((nb, D), lambda i: (i, 0)),
            pl.BlockSpec((nb, D), lambda i: (i, 0)),
            pl.BlockSpec((nb, 1), lambda i: (i, 0)),
            pl.BlockSpec((nb, 1), lambda i: (i, 0)),
            pl.BlockSpec((nb, 1), lambda i: (i, 0)),
            pl.BlockSpec((8, 128), lambda i: (0, 0)),
            pl.BlockSpec((nb, D), lambda i: (i, 0)),
            pl.BlockSpec((1, D), lambda i: (0, 0)),
        ],
        out_specs=pl.BlockSpec((nb, D), lambda i: (i, 0)),
        out_shape=jax.ShapeDtypeStruct((NPAD, D), jnp.float32),
    )(acc2[0], acc2[1], den2[0], den2[1], aloop.reshape(NPAD, 1), gmaxp, h,
      bias)
    return out[:N]


@jax.jit
def kernel(x, edge_index, edge_attr, W, att_src, att_dst, lin_edge_W,
           att_edge, bias):
    src = edge_index[0]
    dst = edge_index[1]
    pad_nodes = (N + (jnp.arange(PAD_E, dtype=jnp.int32) % (NPAD - N)))
    srcp = jnp.concatenate([src, pad_nodes]).reshape(NW, ECH, 128)
    dstp = jnp.concatenate([dst, pad_nodes]).reshape(NW, ECH, 128)
    ea2 = edge_attr.reshape(ER, 128)
    xp = jnp.concatenate([x, jnp.zeros((NPAD - N, D), jnp.float32)], axis=0)
    zeros1 = jnp.zeros((640,), jnp.float32)
    zeros2 = jnp.zeros((640, D), jnp.float32)

    ii = lax.broadcasted_iota(jnp.int32, (128, DE), 0)
    kk = lax.broadcasted_iota(jnp.int32, (128, DE), 1)
    emat = ((ii % DE) == kk).astype(jnp.float32)
    i8 = lax.broadcasted_iota(jnp.int32, (128, 8), 0)
    j8 = lax.broadcasted_iota(jnp.int32, (128, 8), 1)
    mask8 = ((i8 // DE) == j8).astype(jnp.float32)
    h, asrc, adst, ae = _k0(xp, ea2, W, att_src.reshape(1, D),
                            att_dst.reshape(1, D), lin_edge_W,
                            att_edge.reshape(1, D), emat, mask8)
    asrc = asrc.reshape(NPAD)
    adst = adst.reshape(NPAD)
    aep = jnp.concatenate(
        [ae.reshape(E), jnp.zeros((PAD_E,), jnp.float32)]
    ).reshape(NW, ECH, 128)
    alpha, degp, saep, tmax = _k1(srcp, dstp, aep, asrc, adst, zeros1)
    aloop, gmaxp = _k2(degp, saep, asrc, adst, tmax)
    denomp, acc = _k3(srcp, dstp, alpha, gmaxp, h, zeros1, zeros2)
    return _k4(acc, denomp, aloop, gmaxp, h, bias.reshape(1, D))


# speedup vs baseline: 27.7553x; 1.2298x over previous
import functools

import jax
import jax.numpy as jnp
from jax import lax
from jax.experimental import pallas as pl
from jax.experimental.pallas import tpu as pltpu
from jax.experimental.pallas import tpu_sc as plsc

N = 10000
E = 320000
D = 128
DE = 16

NC = 2
NS = 16
NW = NC * NS
LANE = 16

NPAD = 10240
ECH = 79
ET = ECH * 128
EPAD = NW * ET
PAD_E = EPAD - E

ER = E // 8
EB = ER // 8
NB = NPAD // 8


def _k0_body(x_r, ea_r, w_r, asw_r, adw_r, lew_r, aew_r, emat_r, mask8_r,
             h_r, asrc_r, adst_r, ae_r):
    xb = x_r[...]
    h = lax.dot_general(xb, w_r[...], (((1,), (1,)), ((), ())),
                        preferred_element_type=jnp.float32)
    h_r[...] = h
    asrc_r[...] = jnp.sum(h * asw_r[...], axis=1)[None, None, :]
    adst_r[...] = jnp.sum(h * adw_r[...], axis=1)[None, None, :]
    v = lax.dot_general(aew_r[...], lew_r[...], (((1,), (0,)), ((), ())),
                        preferred_element_type=jnp.float32)
    vcol = lax.dot_general(emat_r[...], v, (((1,), (1,)), ((), ())),
                           preferred_element_type=jnp.float32)
    bmat = mask8_r[...] * vcol
    ae_r[...] = lax.dot_general(ea_r[...], bmat, (((1,), (0,)), ((), ())),
                                preferred_element_type=jnp.float32)


def _k0(xp, ea2, W, att_src, att_dst, lin_edge_W, att_edge, emat, mask8):
    return pl.pallas_call(
        _k0_body,
        grid=(8,),
        in_specs=[
            pl.BlockSpec((NB, D), lambda i: (i, 0)),
            pl.BlockSpec((EB, 128), lambda i: (i, 0)),
            pl.BlockSpec((D, D), lambda i: (0, 0)),
            pl.BlockSpec((1, D), lambda i: (0, 0)),
            pl.BlockSpec((1, D), lambda i: (0, 0)),
            pl.BlockSpec((D, DE), lambda i: (0, 0)),
            pl.BlockSpec((1, D), lambda i: (0, 0)),
            pl.BlockSpec((128, DE), lambda i: (0, 0)),
            pl.BlockSpec((128, 8), lambda i: (0, 0)),
        ],
        out_specs=[
            pl.BlockSpec((NB, D), lambda i: (i, 0)),
            pl.BlockSpec((1, 1, NB), lambda i: (i, 0, 0)),
            pl.BlockSpec((1, 1, NB), lambda i: (i, 0, 0)),
            pl.BlockSpec((EB, 8), lambda i: (i, 0)),
        ],
        out_shape=[
            jax.ShapeDtypeStruct((NPAD, D), jnp.float32),
            jax.ShapeDtypeStruct((8, 1, NB), jnp.float32),
            jax.ShapeDtypeStruct((8, 1, NB), jnp.float32),
            jax.ShapeDtypeStruct((ER, 8), jnp.float32),
        ],
    )(xp, ea2, W, att_src, att_dst, lin_edge_W, att_edge, emat, mask8)


def _k1_body(combp, aep, asrc, adst, zeros1,
             alpha_o, degp_o, saep_o, tmax_o,
             comb_v, dst_v, ae_v, alpha_v, asrc_v, adst_v, ones_v, tmax_v,
             deg_sh, sae_sh):
    cid = lax.axis_index("c")
    sid = lax.axis_index("s")
    wid = sid * NC + cid

    pltpu.sync_copy(combp.at[wid], comb_v)
    pltpu.sync_copy(aep.at[wid], ae_v)
    pltpu.sync_copy(asrc, asrc_v)
    pltpu.sync_copy(adst, adst_v)

    for g in range(8):
        ones_v[pl.ds(g * LANE, LANE)] = jnp.full((LANE,), 1.0, jnp.float32)

    pltpu.sync_copy(zeros1, deg_sh.at[pl.ds(sid * 640, 640)])
    pltpu.sync_copy(zeros1, sae_sh.at[pl.ds(sid * 640, 640)])
    plsc.subcore_barrier()

    def chunk(j, m):
        for g in range(8):
            off = pl.ds(g * LANE, LANE)
            c16 = comb_v[j, off]
            s16 = c16 & 0xFFFF
            d16 = lax.shift_right_logical(c16, 16)
            dst_v[j, off] = d16
            ae16 = ae_v[j, off]
            al = plsc.load_gather(asrc_v, [s16]) \
                + plsc.load_gather(adst_v, [d16]) + ae16
            al = jnp.maximum(al, al * 0.2)
            alpha_v[j, off] = al
            m = jnp.maximum(m, al)
        pltpu.sync_copy(ones_v, deg_sh.at[dst_v.at[j]], add=True)
        pltpu.sync_copy(ae_v.at[j], sae_sh.at[dst_v.at[j]], add=True)
        return m

    m0 = jnp.full((LANE,), -1e30, jnp.float32)
    m = lax.fori_loop(0, ECH, chunk, m0)
    tmax_v[...] = m

    pltpu.sync_copy(alpha_v, alpha_o.at[wid])
    pltpu.sync_copy(tmax_v, tmax_o.at[wid])
    plsc.subcore_barrier()
    base = cid * NPAD + sid * 640
    pltpu.sync_copy(deg_sh.at[pl.ds(sid * 640, 640)],
                    degp_o.at[pl.ds(base, 640)])
    pltpu.sync_copy(sae_sh.at[pl.ds(sid * 640, 640)],
                    saep_o.at[pl.ds(base, 640)])


def _k1(combp, aep, asrc, adst, zeros1):
    mesh = plsc.VectorSubcoreMesh(core_axis_name="c", subcore_axis_name="s",
                                  num_cores=NC, num_subcores=NS)
    f = pl.kernel(
        _k1_body,
        out_type=[
            jax.ShapeDtypeStruct((NW, ECH, 128), jnp.float32),
            jax.ShapeDtypeStruct((NC * NPAD,), jnp.float32),
            jax.ShapeDtypeStruct((NC * NPAD,), jnp.float32),
            jax.ShapeDtypeStruct((NW, LANE), jnp.float32),
        ],
        mesh=mesh,
        compiler_params=pltpu.CompilerParams(needs_layout_passes=False),
        scratch_types=[
            pltpu.VMEM((ECH, 128), jnp.int32),
            pltpu.VMEM((ECH, 128), jnp.int32),
            pltpu.VMEM((ECH, 128), jnp.float32),
            pltpu.VMEM((ECH, 128), jnp.float32),
            pltpu.VMEM((NPAD,), jnp.float32),
            pltpu.VMEM((NPAD,), jnp.float32),
            pltpu.VMEM((128,), jnp.float32),
            pltpu.VMEM((LANE,), jnp.float32),
            pltpu.VMEM_SHARED((NPAD,), jnp.float32),
            pltpu.VMEM_SHARED((NPAD,), jnp.float32),
        ],
    )
    return f(combp, aep, asrc, adst, zeros1)


def _k2_body(degp_r, saep_r, asrc_r, adst_r, tmax_r, aloop_o, gmax_o):
    deg = jnp.sum(degp_r[...], axis=0)
    sae = jnp.sum(saep_r[...], axis=0)
    ae_loop = sae / jnp.maximum(deg, 1.0)
    al = asrc_r[...] + adst_r[...] + ae_loop
    al = jnp.maximum(al, al * 0.2)
    aloop_o[...] = al
    gmax = jnp.maximum(jnp.max(tmax_r[...]), jnp.max(al))
    gmax_o[...] = jnp.full((8, 128), gmax, jnp.float32)


def _k2(degp, saep, asrc, adst, tmax):
    return pl.pallas_call(
        _k2_body,
        out_shape=[
            jax.ShapeDtypeStruct((NPAD,), jnp.float32),
            jax.ShapeDtypeStruct((8, 128), jnp.float32),
        ],
    )(degp.reshape(NC, NPAD), saep.reshape(NC, NPAD), asrc, adst, tmax)


def _k3_body(combp, alpha2, gmaxp, h, zeros1, zeros2,
             denomp_o, acc_o,
             comb_v, src_a, src_b, dst_a, dst_b, exp_a, exp_b,
             rows_a, rows_b, gmax_v,
             denom_sh, acc_sh, sem_a, sem_b):
    cid = lax.axis_index("c")
    sid = lax.axis_index("s")
    wid = sid * NC + cid

    pltpu.sync_copy(combp.at[wid], comb_v)
    pltpu.sync_copy(gmaxp.at[0], gmax_v)

    pltpu.sync_copy(zeros1, denom_sh.at[pl.ds(sid * 640, 640)])
    pltpu.sync_copy(zeros2, acc_sh.at[pl.ds(sid * 640, 640)])
    plsc.subcore_barrier()

    gsp = gmax_v[pl.ds(0, LANE)]
    arow0 = wid * ECH

    def prep(j, src_r, dst_r, exp_r, rows, sem):
        for g in range(8):
            off = pl.ds(g * LANE, LANE)
            c16 = comb_v[j, off]
            src_r[off] = c16 & 0xFFFF
            dst_r[off] = lax.shift_right_logical(c16, 16)
        pltpu.async_copy(alpha2.at[arow0 + j], exp_r, sem)
        pltpu.async_copy(h.at[src_r], rows, sem)

    def process(j, src_r, dst_r, exp_r, rows, sem):
        pltpu.make_async_copy(alpha2.at[arow0 + j], exp_r, sem).wait()
        pltpu.make_async_copy(h.at[src_r], rows, sem).wait()
        for g in range(8):
            off = pl.ds(g * LANE, LANE)
            exp_r[off] = jnp.exp(exp_r[off] - gsp)
        pltpu.sync_copy(exp_r, denom_sh.at[dst_r], add=True)

        def scale(r, c):
            spl = plsc.load_gather(exp_r, [jnp.full((LANE,), r, jnp.int32)])
            for g in range(8):
                off = pl.ds(g * LANE, LANE)
                rows[r, off] = rows[r, off] * spl
            return c
        lax.fori_loop(0, 128, scale, 0)
        pltpu.sync_copy(rows, acc_sh.at[dst_r], add=True)

    prep(0, src_a, dst_a, exp_a, rows_a, sem_a)

    def pair(j2, carry):
        ja = 2 * j2
        jb = ja + 1
        prep(jb, src_b, dst_b, exp_b, rows_b, sem_b)
        process(ja, src_a, dst_a, exp_a, rows_a, sem_a)
        prep(ja + 2, src_a, dst_a, exp_a, rows_a, sem_a)
        process(jb, src_b, dst_b, exp_b, rows_b, sem_b)
        return carry

    lax.fori_loop(0, (ECH - 1) // 2, pair, 0)
    process(ECH - 1, src_a, dst_a, exp_a, rows_a, sem_a)

    plsc.subcore_barrier()
    base = cid * NPAD + sid * 640
    pltpu.sync_copy(denom_sh.at[pl.ds(sid * 640, 640)],
                    denomp_o.at[pl.ds(base, 640)])
    pltpu.sync_copy(acc_sh.at[pl.ds(sid * 640, 640)],
                    acc_o.at[pl.ds(base, 640)])


def _k3(combp, alpha2, gmaxp, h, zeros1, zeros2):
    mesh = plsc.VectorSubcoreMesh(core_axis_name="c", subcore_axis_name="s",
                                  num_cores=NC, num_subcores=NS)
    f = pl.kernel(
        _k3_body,
        out_type=[
            jax.ShapeDtypeStruct((NC * NPAD,), jnp.float32),
            jax.ShapeDtypeStruct((NC * NPAD, D), jnp.float32),
        ],
        mesh=mesh,
        compiler_params=pltpu.CompilerParams(needs_layout_passes=False),
        scratch_types=[
            pltpu.VMEM((ECH, 128), jnp.int32),
            pltpu.VMEM((128,), jnp.int32),
            pltpu.VMEM((128,), jnp.int32),
            pltpu.VMEM((128,), jnp.int32),
            pltpu.VMEM((128,), jnp.int32),
            pltpu.VMEM((128,), jnp.float32),
            pltpu.VMEM((128,), jnp.float32),
            pltpu.VMEM((128, D), jnp.float32),
            pltpu.VMEM((128, D), jnp.float32),
            pltpu.VMEM((128,), jnp.float32),
            pltpu.VMEM_SHARED((NPAD,), jnp.float32),
            pltpu.VMEM_SHARED((NPAD, D), jnp.float32),
            pltpu.SemaphoreType.DMA,
            pltpu.SemaphoreType.DMA,
        ],
    )
    return f(combp, alpha2, gmaxp, h, zeros1, zeros2)


def _k4_body(acc0_r, acc1_r, den0_r, den1_r, aloop_r, gmax_r, h_r, bias_r,
             out_r):
    gm = jnp.max(gmax_r[...])
    el = jnp.exp(aloop_r[...] - gm)
    den = den0_r[...] + den1_r[...] + el + 1e-16
    num = acc0_r[...] + acc1_r[...] + el * h_r[...]
    out = num / den + bias_r[...]
    out_r[...] = jnp.maximum(out, 0.0)


def _k4(acc, denomp, aloop, gmaxp, h, bias):
    acc2 = acc.reshape(NC, NPAD, D)
    den2 = denomp.reshape(NC, NPAD, 1)
    nb = NPAD // 8
    out = pl.pallas_call(
        _k4_body,
        grid=(8,),
        in_specs=[
            pl.BlockSpec((nb, D), lambda i: (i, 0)),
            pl.BlockSpec((nb, D), lambda i: (i, 0)),
            pl.BlockSpec((nb, 1), lambda i: (i, 0)),
            pl.BlockSpec((nb, 1), lambda i: (i, 0)),
            pl.BlockSpec((nb, 1), lambda i: (i, 0)),
            pl.BlockSpec((8, 128), lambda i: (0, 0)),
            pl.BlockSpec((nb, D), lambda i: (i, 0)),
            pl.BlockSpec((1, D), lambda i: (0, 0)),
        ],
        out_specs=pl.BlockSpec((nb, D), lambda i: (i, 0)),
        out_shape=jax.ShapeDtypeStruct((NPAD, D), jnp.float32),
    )(acc2[0], acc2[1], den2[0], den2[1], aloop.reshape(NPAD, 1), gmaxp, h,
      bias)
    return out[:N]


@jax.jit
def kernel(x, edge_index, edge_attr, W, att_src, att_dst, lin_edge_W,
           att_edge, bias):
    src = edge_index[0]
    dst = edge_index[1]
    pad_nodes = (N + (jnp.arange(PAD_E, dtype=jnp.int32) % (NPAD - N)))
    comb = src | (dst << 16)
    combp = jnp.concatenate(
        [comb, pad_nodes | (pad_nodes << 16)]).reshape(NW, ECH, 128)
    ea2 = edge_attr.reshape(ER, 128)
    xp = jnp.concatenate([x, jnp.zeros((NPAD - N, D), jnp.float32)], axis=0)
    zeros1 = jnp.zeros((640,), jnp.float32)
    zeros2 = jnp.zeros((640, D), jnp.float32)

    ii = lax.broadcasted_iota(jnp.int32, (128, DE), 0)
    kk = lax.broadcasted_iota(jnp.int32, (128, DE), 1)
    emat = ((ii % DE) == kk).astype(jnp.float32)
    i8 = lax.broadcasted_iota(jnp.int32, (128, 8), 0)
    j8 = lax.broadcasted_iota(jnp.int32, (128, 8), 1)
    mask8 = ((i8 // DE) == j8).astype(jnp.float32)
    h, asrc, adst, ae = _k0(xp, ea2, W, att_src.reshape(1, D),
                            att_dst.reshape(1, D), lin_edge_W,
                            att_edge.reshape(1, D), emat, mask8)
    asrc = asrc.reshape(NPAD)
    adst = adst.reshape(NPAD)
    aep = jnp.concatenate(
        [ae.reshape(E), jnp.zeros((PAD_E,), jnp.float32)]
    ).reshape(NW, ECH, 128)
    alpha, degp, saep, tmax = _k1(combp, aep, asrc, adst, zeros1)
    aloop, gmaxp = _k2(degp, saep, asrc, adst, tmax)
    denomp, acc = _k3(combp, alpha.reshape(NW * ECH, 128), gmaxp, h,
                      zeros1, zeros2)
    return _k4(acc, denomp, aloop, gmaxp, h, bias.reshape(1, D))


# speedup vs baseline: 28.0444x; 1.0104x over previous
import functools

import jax
import jax.numpy as jnp
from jax import lax
from jax.experimental import pallas as pl
from jax.experimental.pallas import tpu as pltpu
from jax.experimental.pallas import tpu_sc as plsc

N = 10000
E = 320000
D = 128
DE = 16

NC = 2
NS = 16
NW = NC * NS
LANE = 16

NPAD = 10240
ECH = 79
ET = ECH * 128
EPAD = NW * ET
PAD_E = EPAD - E
EROWS = NW * ECH

ER = E // 8
EB = ER // 8
NB = NPAD // 8


def _k0a_body(x_r, w_r, asw_r, adw_r, h_r, asrc_r, adst_r):
    xb = x_r[...]
    h = lax.dot_general(xb, w_r[...], (((1,), (1,)), ((), ())),
                        preferred_element_type=jnp.float32)
    h_r[...] = h
    asrc_r[...] = jnp.sum(h * asw_r[...], axis=1)[None, None, :]
    adst_r[...] = jnp.sum(h * adw_r[...], axis=1)[None, None, :]


def _k0a(xp, W, att_src, att_dst):
    return pl.pallas_call(
        _k0a_body,
        grid=(8,),
        in_specs=[
            pl.BlockSpec((NB, D), lambda i: (i, 0)),
            pl.BlockSpec((D, D), lambda i: (0, 0)),
            pl.BlockSpec((1, D), lambda i: (0, 0)),
            pl.BlockSpec((1, D), lambda i: (0, 0)),
        ],
        out_specs=[
            pl.BlockSpec((NB, D), lambda i: (i, 0)),
            pl.BlockSpec((1, 1, NB), lambda i: (i, 0, 0)),
            pl.BlockSpec((1, 1, NB), lambda i: (i, 0, 0)),
        ],
        out_shape=[
            jax.ShapeDtypeStruct((NPAD, D), jnp.float32),
            jax.ShapeDtypeStruct((8, 1, NB), jnp.float32),
            jax.ShapeDtypeStruct((8, 1, NB), jnp.float32),
        ],
    )(xp, W, att_src, att_dst)


def _k0b_body(ea_r, lew_r, aew_r, emat_r, mask8_r, ae_r):
    v = lax.dot_general(aew_r[...], lew_r[...], (((1,), (0,)), ((), ())),
                        preferred_element_type=jnp.float32)
    vcol = lax.dot_general(emat_r[...], v, (((1,), (1,)), ((), ())),
                           preferred_element_type=jnp.float32)
    bmat = mask8_r[...] * vcol
    ae_r[...] = lax.dot_general(ea_r[...], bmat, (((1,), (0,)), ((), ())),
                                preferred_element_type=jnp.float32)


def _k0b(ea2, lin_edge_W, att_edge, emat, mask8):
    return pl.pallas_call(
        _k0b_body,
        grid=(8,),
        in_specs=[
            pl.BlockSpec((EB, 128), lambda i: (i, 0)),
            pl.BlockSpec((D, DE), lambda i: (0, 0)),
            pl.BlockSpec((1, D), lambda i: (0, 0)),
            pl.BlockSpec((128, DE), lambda i: (0, 0)),
            pl.BlockSpec((128, 8), lambda i: (0, 0)),
        ],
        out_specs=pl.BlockSpec((EB, 8), lambda i: (i, 0)),
        out_shape=jax.ShapeDtypeStruct((ER, 8), jnp.float32),
    )(ea2, lin_edge_W, att_edge, emat, mask8)


def _k1a_body(combp, asrc, adst, zeros1,
              asum_o, degp_o, dstp_o,
              comb_v, dst_v, asum_v, asrc_v, adst_v, ones_v,
              deg_sh):
    cid = lax.axis_index("c")
    sid = lax.axis_index("s")
    wid = sid * NC + cid

    pltpu.sync_copy(combp.at[wid], comb_v)
    pltpu.sync_copy(asrc, asrc_v)
    pltpu.sync_copy(adst, adst_v)

    for g in range(8):
        ones_v[pl.ds(g * LANE, LANE)] = jnp.full((LANE,), 1.0, jnp.float32)

    pltpu.sync_copy(zeros1, deg_sh.at[pl.ds(sid * 640, 640)])
    plsc.subcore_barrier()

    def chunk(j, c):
        for g in range(8):
            off = pl.ds(g * LANE, LANE)
            c16 = comb_v[j, off]
            s16 = c16 & 0xFFFF
            d16 = lax.shift_right_logical(c16, 16)
            dst_v[j, off] = d16
            asum_v[j, off] = plsc.load_gather(asrc_v, [s16]) \
                + plsc.load_gather(adst_v, [d16])
        pltpu.sync_copy(ones_v, deg_sh.at[dst_v.at[j]], add=True)
        return c

    lax.fori_loop(0, ECH, chunk, 0)

    pltpu.sync_copy(asum_v, asum_o.at[wid])
    pltpu.sync_copy(dst_v, dstp_o.at[wid])
    plsc.subcore_barrier()
    base = cid * NPAD + sid * 640
    pltpu.sync_copy(deg_sh.at[pl.ds(sid * 640, 640)],
                    degp_o.at[pl.ds(base, 640)])


def _k1a(combp, asrc, adst, zeros1):
    mesh = plsc.VectorSubcoreMesh(core_axis_name="c", subcore_axis_name="s",
                                  num_cores=NC, num_subcores=NS)
    f = pl.kernel(
        _k1a_body,
        out_type=[
            jax.ShapeDtypeStruct((NW, ECH, 128), jnp.float32),
            jax.ShapeDtypeStruct((NC * NPAD,), jnp.float32),
            jax.ShapeDtypeStruct((NW, ECH, 128), jnp.int32),
        ],
        mesh=mesh,
        compiler_params=pltpu.CompilerParams(needs_layout_passes=False),
        scratch_types=[
            pltpu.VMEM((ECH, 128), jnp.int32),
            pltpu.VMEM((ECH, 128), jnp.int32),
            pltpu.VMEM((ECH, 128), jnp.float32),
            pltpu.VMEM((NPAD,), jnp.float32),
            pltpu.VMEM((NPAD,), jnp.float32),
            pltpu.VMEM((128,), jnp.float32),
            pltpu.VMEM_SHARED((NPAD,), jnp.float32),
        ],
    )
    return f(combp, asrc, adst, zeros1)


def _k1b_body(dstp, aef, zeros1,
              saep_o,
              dst_v, ae_v, sae_sh):
    cid = lax.axis_index("c")
    sid = lax.axis_index("s")
    wid = sid * NC + cid

    pltpu.sync_copy(dstp.at[wid], dst_v)
    pltpu.sync_copy(aef.at[pl.ds(wid * ET, ET)], ae_v)
    pltpu.sync_copy(zeros1, sae_sh.at[pl.ds(sid * 640, 640)])
    plsc.subcore_barrier()

    def chunk(j, c):
        pltpu.sync_copy(ae_v.at[pl.ds(j * 128, 128)],
                        sae_sh.at[dst_v.at[j]], add=True)
        return c

    lax.fori_loop(0, ECH, chunk, 0)

    plsc.subcore_barrier()
    base = cid * NPAD + sid * 640
    pltpu.sync_copy(sae_sh.at[pl.ds(sid * 640, 640)],
                    saep_o.at[pl.ds(base, 640)])


def _k1b(dstp, aef, zeros1):
    mesh = plsc.VectorSubcoreMesh(core_axis_name="c", subcore_axis_name="s",
                                  num_cores=NC, num_subcores=NS)
    f = pl.kernel(
        _k1b_body,
        out_type=[
            jax.ShapeDtypeStruct((NC * NPAD,), jnp.float32),
        ],
        mesh=mesh,
        compiler_params=pltpu.CompilerParams(needs_layout_passes=False),
        scratch_types=[
            pltpu.VMEM((ECH, 128), jnp.int32),
            pltpu.VMEM((ET,), jnp.float32),
            pltpu.VMEM_SHARED((NPAD,), jnp.float32),
        ],
    )
    return f(dstp, aef, zeros1)


def _k2_body(asum_r, ae_r, alpha_o, gmax_o):
    al = asum_r[...] + ae_r[...]
    al = jnp.maximum(al, al * 0.2)
    alpha_o[...] = al
    m = jnp.max(al)
    i = pl.program_id(0)

    @pl.when(i == 0)
    def _():
        gmax_o[...] = jnp.full((8, 128), m, jnp.float32)

    @pl.when(i > 0)
    def _():
        gmax_o[...] = jnp.maximum(gmax_o[...], m)


def _k2(asum, ae2):
    kb = EROWS // 4
    return pl.pallas_call(
        _k2_body,
        grid=(4,),
        in_specs=[
            pl.BlockSpec((kb, 128), lambda i: (i, 0)),
            pl.BlockSpec((kb, 128), lambda i: (i, 0)),
        ],
        out_specs=[
            pl.BlockSpec((kb, 128), lambda i: (i, 0)),
            pl.BlockSpec((8, 128), lambda i: (0, 0)),
        ],
        out_shape=[
            jax.ShapeDtypeStruct((EROWS, 128), jnp.float32),
            jax.ShapeDtypeStruct((8, 128), jnp.float32),
        ],
    )(asum, ae2)


def _k3_body(combp, alpha2, gmaxp, h, zeros1, zeros2,
             denomp_o, acc_o,
             comb_v, src_a, src_b, dst_a, dst_b, exp_a, exp_b,
             rows_a, rows_b, gmax_v,
             denom_sh, acc_sh, sem_a, sem_b):
    cid = lax.axis_index("c")
    sid = lax.axis_index("s")
    wid = sid * NC + cid

    pltpu.sync_copy(combp.at[wid], comb_v)
    pltpu.sync_copy(gmaxp.at[0], gmax_v)

    pltpu.sync_copy(zeros1, denom_sh.at[pl.ds(sid * 640, 640)])
    pltpu.sync_copy(zeros2, acc_sh.at[pl.ds(sid * 640, 640)])
    plsc.subcore_barrier()

    gsp = gmax_v[pl.ds(0, LANE)]
    arow0 = wid * ECH

    def prep(j, src_r, dst_r, exp_r, rows, sem):
        for g in range(8):
            off = pl.ds(g * LANE, LANE)
            c16 = comb_v[j, off]
            src_r[off] = c16 & 0xFFFF
            dst_r[off] = lax.shift_right_logical(c16, 16)
        pltpu.async_copy(alpha2.at[arow0 + j], exp_r, sem)
        pltpu.async_copy(h.at[src_r], rows, sem)

    def process(j, src_r, dst_r, exp_r, rows, sem):
        pltpu.make_async_copy(alpha2.at[arow0 + j], exp_r, sem).wait()
        pltpu.make_async_copy(h.at[src_r], rows, sem).wait()
        for g in range(8):
            off = pl.ds(g * LANE, LANE)
            exp_r[off] = jnp.exp(exp_r[off] - gsp)
        pltpu.sync_copy(exp_r, denom_sh.at[dst_r], add=True)

        def scale(r, c):
            spl = plsc.load_gather(exp_r, [jnp.full((LANE,), r, jnp.int32)])
            for g in range(8):
                off = pl.ds(g * LANE, LANE)
                rows[r, off] = rows[r, off] * spl
            return c
        lax.fori_loop(0, 128, scale, 0)
        pltpu.sync_copy(rows, acc_sh.at[dst_r], add=True)

    prep(0, src_a, dst_a, exp_a, rows_a, sem_a)

    def pair(j2, carry):
        ja = 2 * j2
        jb = ja + 1
        prep(jb, src_b, dst_b, exp_b, rows_b, sem_b)
        process(ja, src_a, dst_a, exp_a, rows_a, sem_a)
        prep(ja + 2, src_a, dst_a, exp_a, rows_a, sem_a)
        process(jb, src_b, dst_b, exp_b, rows_b, sem_b)
        return carry

    lax.fori_loop(0, (ECH - 1) // 2, pair, 0)
    process(ECH - 1, src_a, dst_a, exp_a, rows_a, sem_a)

    plsc.subcore_barrier()
    base = cid * NPAD + sid * 640
    pltpu.sync_copy(denom_sh.at[pl.ds(sid * 640, 640)],
                    denomp_o.at[pl.ds(base, 640)])
    pltpu.sync_copy(acc_sh.at[pl.ds(sid * 640, 640)],
                    acc_o.at[pl.ds(base, 640)])


def _k3(combp, alpha2, gmaxp, h, zeros1, zeros2):
    mesh = plsc.VectorSubcoreMesh(core_axis_name="c", subcore_axis_name="s",
                                  num_cores=NC, num_subcores=NS)
    f = pl.kernel(
        _k3_body,
        out_type=[
            jax.ShapeDtypeStruct((NC * NPAD,), jnp.float32),
            jax.ShapeDtypeStruct((NC * NPAD, D), jnp.float32),
        ],
        mesh=mesh,
        compiler_params=pltpu.CompilerParams(needs_layout_passes=False),
        scratch_types=[
            pltpu.VMEM((ECH, 128), jnp.int32),
            pltpu.VMEM((128,), jnp.int32),
            pltpu.VMEM((128,), jnp.int32),
            pltpu.VMEM((128,), jnp.int32),
            pltpu.VMEM((128,), jnp.int32),
            pltpu.VMEM((128,), jnp.float32),
            pltpu.VMEM((128,), jnp.float32),
            pltpu.VMEM((128, D), jnp.float32),
            pltpu.VMEM((128, D), jnp.float32),
            pltpu.VMEM((128,), jnp.float32),
            pltpu.VMEM_SHARED((NPAD,), jnp.float32),
            pltpu.VMEM_SHARED((NPAD, D), jnp.float32),
            pltpu.SemaphoreType.DMA,
            pltpu.SemaphoreType.DMA,
        ],
    )
    return f(combp, alpha2, gmaxp, h, zeros1, zeros2)


def _k4_body(acc0_r, acc1_r, den0_r, den1_r, sae0_r, sae1_r,
             deg0_r, deg1_r, asrc_r, adst_r, gmax_r, h_r, bias_r,
             out_r):
    gm = jnp.max(gmax_r[...])
    deg = deg0_r[...] + deg1_r[...]
    sae = sae0_r[...] + sae1_r[...]
    aloop = asrc_r[...] + adst_r[...] + sae / jnp.maximum(deg, 1.0)
    aloop = jnp.maximum(aloop, aloop * 0.2)
    el = jnp.exp(aloop - gm)
    den = den0_r[...] + den1_r[...] + el + 1e-16
    num = acc0_r[...] + acc1_r[...] + el * h_r[...]
    out = num / den + bias_r[...]
    out_r[...] = jnp.maximum(out, 0.0)


def _k4(acc, denomp, saep, degp, asrc, adst, gmaxp, h, bias):
    nb = NPAD // 8
    nblk = NPAD // nb

    def lo(i):
        return (i, 0)

    def hi(i):
        return (i + nblk, 0)

    den2 = denomp.reshape(NC * NPAD, 1)
    sae2 = saep.reshape(NC * NPAD, 1)
    deg2 = degp.reshape(NC * NPAD, 1)
    return pl.pallas_call(
        _k4_body,
        grid=(8,),
        in_specs=[
            pl.BlockSpec((nb, D), lo),
            pl.BlockSpec((nb, D), hi),
            pl.BlockSpec((nb, 1), lo),
            pl.BlockSpec((nb, 1), hi),
            pl.BlockSpec((nb, 1), lo),
            pl.BlockSpec((nb, 1), hi),
            pl.BlockSpec((nb, 1), lo),
            pl.BlockSpec((nb, 1), hi),
            pl.BlockSpec((nb, 1), lo),
            pl.BlockSpec((nb, 1), lo),
            pl.BlockSpec((8, 128), lambda i: (0, 0)),
            pl.BlockSpec((nb, D), lo),
            pl.BlockSpec((1, D), lambda i: (0, 0)),
        ],
        out_specs=pl.BlockSpec((nb, D), lo),
        out_shape=jax.ShapeDtypeStruct((N, D), jnp.float32),
    )(acc, acc, den2, den2, sae2, sae2, deg2, deg2,
      asrc.reshape(NPAD, 1), adst.reshape(NPAD, 1), gmaxp, h, bias)


@jax.jit
def kernel(x, edge_index, edge_attr, W, att_src, att_dst, lin_edge_W,
           att_edge, bias):
    src = edge_index[0]
    dst = edge_index[1]
    pad_nodes = (N + (jnp.arange(PAD_E, dtype=jnp.int32) % (NPAD - N)))
    comb = src | (dst << 16)
    combp = jnp.concatenate(
        [comb, pad_nodes | (pad_nodes << 16)]).reshape(NW, ECH, 128)
    ea2 = edge_attr.reshape(ER, 128)
    xp = jnp.concatenate([x, jnp.zeros((NPAD - N, D), jnp.float32)], axis=0)
    zeros1 = jnp.zeros((640,), jnp.float32)
    zeros2 = jnp.zeros((640, D), jnp.float32)
    ii = lax.broadcasted_iota(jnp.int32, (128, DE), 0)
    kk = lax.broadcasted_iota(jnp.int32, (128, DE), 1)
    emat = ((ii % DE) == kk).astype(jnp.float32)
    i8 = lax.broadcasted_iota(jnp.int32, (128, 8), 0)
    j8 = lax.broadcasted_iota(jnp.int32, (128, 8), 1)
    mask8 = ((i8 // DE) == j8).astype(jnp.float32)

    h, asrc3, adst3 = _k0a(xp, W, att_src.reshape(1, D),
                           att_dst.reshape(1, D))
    asrc = asrc3.reshape(NPAD)
    adst = adst3.reshape(NPAD)
    asum, degp, dstp = _k1a(combp, asrc, adst, zeros1)

    ae = _k0b(ea2, lin_edge_W, att_edge.reshape(1, D), emat, mask8)
    aef = jnp.concatenate(
        [ae.reshape(E), jnp.zeros((PAD_E,), jnp.float32)])
    saep, = _k1b(dstp, aef, zeros1)

    alpha2, gmaxp = _k2(asum.reshape(EROWS, 128), aef.reshape(EROWS, 128))
    denomp, acc = _k3(combp, alpha2, gmaxp, h, zeros1, zeros2)
    return _k4(acc, denomp, saep, degp, asrc, adst, gmaxp, h,
               bias.reshape(1, D))


# speedup vs baseline: 28.3118x; 1.0095x over previous
import functools

import jax
import jax.numpy as jnp
from jax import lax
from jax.experimental import pallas as pl
from jax.experimental.pallas import tpu as pltpu
from jax.experimental.pallas import tpu_sc as plsc

N = 10000
E = 320000
D = 128
DE = 16

NC = 2
NS = 16
NW = NC * NS
LANE = 16

NPAD = 10240
ECH = 79
ET = ECH * 128
EPAD = NW * ET
PAD_E = EPAD - E
EROWS = NW * ECH

ER = E // 8
EB = ER // 8
NB = NPAD // 8


def _k0a_body(x_r, w_r, asw_r, adw_r, h_r, asrc_r, adst_r):
    xb = x_r[...]
    h = lax.dot_general(xb, w_r[...], (((1,), (1,)), ((), ())),
                        preferred_element_type=jnp.float32)
    h_r[...] = h
    asrc_r[...] = jnp.sum(h * asw_r[...], axis=1)[None, None, :]
    adst_r[...] = jnp.sum(h * adw_r[...], axis=1)[None, None, :]


def _k0a(xp, W, att_src, att_dst):
    return pl.pallas_call(
        _k0a_body,
        grid=(8,),
        in_specs=[
            pl.BlockSpec((NB, D), lambda i: (i, 0)),
            pl.BlockSpec((D, D), lambda i: (0, 0)),
            pl.BlockSpec((1, D), lambda i: (0, 0)),
            pl.BlockSpec((1, D), lambda i: (0, 0)),
        ],
        out_specs=[
            pl.BlockSpec((NB, D), lambda i: (i, 0)),
            pl.BlockSpec((1, 1, NB), lambda i: (i, 0, 0)),
            pl.BlockSpec((1, 1, NB), lambda i: (i, 0, 0)),
        ],
        out_shape=[
            jax.ShapeDtypeStruct((NPAD, D), jnp.float32),
            jax.ShapeDtypeStruct((8, 1, NB), jnp.float32),
            jax.ShapeDtypeStruct((8, 1, NB), jnp.float32),
        ],
    )(xp, W, att_src, att_dst)


def _k0b_body(ea_r, lew_r, aew_r, emat_r, mask8_r, ae_r):
    v = lax.dot_general(aew_r[...], lew_r[...], (((1,), (0,)), ((), ())),
                        preferred_element_type=jnp.float32)
    vcol = lax.dot_general(emat_r[...], v, (((1,), (1,)), ((), ())),
                           preferred_element_type=jnp.float32)
    bmat = mask8_r[...] * vcol
    ae_r[...] = lax.dot_general(ea_r[...], bmat, (((1,), (0,)), ((), ())),
                                preferred_element_type=jnp.float32)


def _k0b(ea2, lin_edge_W, att_edge, emat, mask8):
    return pl.pallas_call(
        _k0b_body,
        grid=(8,),
        in_specs=[
            pl.BlockSpec((EB, 128), lambda i: (i, 0)),
            pl.BlockSpec((D, DE), lambda i: (0, 0)),
            pl.BlockSpec((1, D), lambda i: (0, 0)),
            pl.BlockSpec((128, DE), lambda i: (0, 0)),
            pl.BlockSpec((128, 8), lambda i: (0, 0)),
        ],
        out_specs=pl.BlockSpec((EB, 8), lambda i: (i, 0)),
        out_shape=jax.ShapeDtypeStruct((ER, 8), jnp.float32),
    )(ea2, lin_edge_W, att_edge, emat, mask8)


def _k1a_body(combp, asrc, adst, zeros1,
              asum_o, degp_o,
              comb_v, dst_v, asum_v, asrc_v, adst_v, ones_v,
              deg_sh):
    cid = lax.axis_index("c")
    sid = lax.axis_index("s")
    wid = sid * NC + cid

    pltpu.sync_copy(combp.at[wid], comb_v)
    pltpu.sync_copy(asrc, asrc_v)
    pltpu.sync_copy(adst, adst_v)

    for g in range(8):
        ones_v[pl.ds(g * LANE, LANE)] = jnp.full((LANE,), 1.0, jnp.float32)

    pltpu.sync_copy(zeros1, deg_sh.at[pl.ds(sid * 640, 640)])
    plsc.subcore_barrier()

    def chunk(j, c):
        for g in range(8):
            off = pl.ds(g * LANE, LANE)
            c16 = comb_v[j, off]
            s16 = c16 & 0xFFFF
            d16 = lax.shift_right_logical(c16, 16)
            dst_v[j, off] = d16
            asum_v[j, off] = plsc.load_gather(asrc_v, [s16]) \
                + plsc.load_gather(adst_v, [d16])
        pltpu.sync_copy(ones_v, deg_sh.at[dst_v.at[j]], add=True)
        return c

    lax.fori_loop(0, ECH, chunk, 0)

    pltpu.sync_copy(asum_v, asum_o.at[wid])
    plsc.subcore_barrier()
    base = cid * NPAD + sid * 640
    pltpu.sync_copy(deg_sh.at[pl.ds(sid * 640, 640)],
                    degp_o.at[pl.ds(base, 640)])


def _k1a(combp, asrc, adst, zeros1):
    mesh = plsc.VectorSubcoreMesh(core_axis_name="c", subcore_axis_name="s",
                                  num_cores=NC, num_subcores=NS)
    f = pl.kernel(
        _k1a_body,
        out_type=[
            jax.ShapeDtypeStruct((NW, ECH, 128), jnp.float32),
            jax.ShapeDtypeStruct((NC * NPAD,), jnp.float32),
        ],
        mesh=mesh,
        compiler_params=pltpu.CompilerParams(needs_layout_passes=False),
        scratch_types=[
            pltpu.VMEM((ECH, 128), jnp.int32),
            pltpu.VMEM((ECH, 128), jnp.int32),
            pltpu.VMEM((ECH, 128), jnp.float32),
            pltpu.VMEM((NPAD,), jnp.float32),
            pltpu.VMEM((NPAD,), jnp.float32),
            pltpu.VMEM((128,), jnp.float32),
            pltpu.VMEM_SHARED((NPAD,), jnp.float32),
        ],
    )
    return f(combp, asrc, adst, zeros1)


def _k2_body(asum_r, ae_r, alpha_o, gmax_o):
    al = asum_r[...] + ae_r[...]
    al = jnp.maximum(al, al * 0.2)
    alpha_o[...] = al
    m = jnp.max(al)
    i = pl.program_id(0)

    @pl.when(i == 0)
    def _():
        gmax_o[...] = jnp.full((8, 128), m, jnp.float32)

    @pl.when(i > 0)
    def _():
        gmax_o[...] = jnp.maximum(gmax_o[...], m)


def _k2(asum, ae2):
    kb = EROWS // 4
    return pl.pallas_call(
        _k2_body,
        grid=(4,),
        in_specs=[
            pl.BlockSpec((kb, 128), lambda i: (i, 0)),
            pl.BlockSpec((kb, 128), lambda i: (i, 0)),
        ],
        out_specs=[
            pl.BlockSpec((kb, 128), lambda i: (i, 0)),
            pl.BlockSpec((8, 128), lambda i: (0, 0)),
        ],
        out_shape=[
            jax.ShapeDtypeStruct((EROWS, 128), jnp.float32),
            jax.ShapeDtypeStruct((8, 128), jnp.float32),
        ],
    )(asum, ae2)


def _k3_body(combp, alpha2, aerows, gmaxp, h, zeros1, zeros2,
             denomp_o, saep_o, acc_o,
             comb_v, src_a, src_b, dst_a, dst_b, exp_a, exp_b,
             ae_a, ae_b, rows_a, rows_b, gmax_v,
             denom_sh, sae_sh, acc_sh, sem_a, sem_b):
    cid = lax.axis_index("c")
    sid = lax.axis_index("s")
    wid = sid * NC + cid

    pltpu.sync_copy(combp.at[wid], comb_v)
    pltpu.sync_copy(gmaxp.at[0], gmax_v)

    pltpu.sync_copy(zeros1, denom_sh.at[pl.ds(sid * 640, 640)])
    pltpu.sync_copy(zeros1, sae_sh.at[pl.ds(sid * 640, 640)])
    pltpu.sync_copy(zeros2, acc_sh.at[pl.ds(sid * 640, 640)])
    plsc.subcore_barrier()

    gsp = gmax_v[pl.ds(0, LANE)]
    arow0 = wid * ECH

    def prep(j, src_r, dst_r, exp_r, ae_r, rows, sem):
        for g in range(8):
            off = pl.ds(g * LANE, LANE)
            c16 = comb_v[j, off]
            src_r[off] = c16 & 0xFFFF
            dst_r[off] = lax.shift_right_logical(c16, 16)
        pltpu.async_copy(alpha2.at[arow0 + j], exp_r, sem)
        pltpu.async_copy(aerows.at[arow0 + j], ae_r, sem)
        pltpu.async_copy(h.at[src_r], rows, sem)

    def process(j, src_r, dst_r, exp_r, ae_r, rows, sem):
        pltpu.make_async_copy(alpha2.at[arow0 + j], exp_r, sem).wait()
        pltpu.make_async_copy(aerows.at[arow0 + j], ae_r, sem).wait()
        pltpu.make_async_copy(h.at[src_r], rows, sem).wait()
        for g in range(8):
            off = pl.ds(g * LANE, LANE)
            exp_r[off] = jnp.exp(exp_r[off] - gsp)
        pltpu.sync_copy(exp_r, denom_sh.at[dst_r], add=True)
        pltpu.sync_copy(ae_r, sae_sh.at[dst_r], add=True)

        def scale(r, c):
            spl = plsc.load_gather(exp_r, [jnp.full((LANE,), r, jnp.int32)])
            for g in range(8):
                off = pl.ds(g * LANE, LANE)
                rows[r, off] = rows[r, off] * spl
            return c
        lax.fori_loop(0, 128, scale, 0)
        pltpu.sync_copy(rows, acc_sh.at[dst_r], add=True)

    prep(0, src_a, dst_a, exp_a, ae_a, rows_a, sem_a)

    def pair(j2, carry):
        ja = 2 * j2
        jb = ja + 1
        prep(jb, src_b, dst_b, exp_b, ae_b, rows_b, sem_b)
        process(ja, src_a, dst_a, exp_a, ae_a, rows_a, sem_a)
        prep(ja + 2, src_a, dst_a, exp_a, ae_a, rows_a, sem_a)
        process(jb, src_b, dst_b, exp_b, ae_b, rows_b, sem_b)
        return carry

    lax.fori_loop(0, (ECH - 1) // 2, pair, 0)
    process(ECH - 1, src_a, dst_a, exp_a, ae_a, rows_a, sem_a)

    plsc.subcore_barrier()
    base = cid * NPAD + sid * 640
    pltpu.sync_copy(denom_sh.at[pl.ds(sid * 640, 640)],
                    denomp_o.at[pl.ds(base, 640)])
    pltpu.sync_copy(sae_sh.at[pl.ds(sid * 640, 640)],
                    saep_o.at[pl.ds(base, 640)])
    pltpu.sync_copy(acc_sh.at[pl.ds(sid * 640, 640)],
                    acc_o.at[pl.ds(base, 640)])


def _k3(combp, alpha2, aerows, gmaxp, h, zeros1, zeros2):
    mesh = plsc.VectorSubcoreMesh(core_axis_name="c", subcore_axis_name="s",
                                  num_cores=NC, num_subcores=NS)
    f = pl.kernel(
        _k3_body,
        out_type=[
            jax.ShapeDtypeStruct((NC * NPAD,), jnp.float32),
            jax.ShapeDtypeStruct((NC * NPAD,), jnp.float32),
            jax.ShapeDtypeStruct((NC * NPAD, D), jnp.float32),
        ],
        mesh=mesh,
        compiler_params=pltpu.CompilerParams(needs_layout_passes=False),
        scratch_types=[
            pltpu.VMEM((ECH, 128), jnp.int32),
            pltpu.VMEM((128,), jnp.int32),
            pltpu.VMEM((128,), jnp.int32),
            pltpu.VMEM((128,), jnp.int32),
            pltpu.VMEM((128,), jnp.int32),
            pltpu.VMEM((128,), jnp.float32),
            pltpu.VMEM((128,), jnp.float32),
            pltpu.VMEM((128,), jnp.float32),
            pltpu.VMEM((128,), jnp.float32),
            pltpu.VMEM((128, D), jnp.float32),
            pltpu.VMEM((128, D), jnp.float32),
            pltpu.VMEM((128,), jnp.float32),
            pltpu.VMEM_SHARED((NPAD,), jnp.float32),
            pltpu.VMEM_SHARED((NPAD,), jnp.float32),
            pltpu.VMEM_SHARED((NPAD, D), jnp.float32),
            pltpu.SemaphoreType.DMA,
            pltpu.SemaphoreType.DMA,
        ],
    )
    return f(combp, alpha2, aerows, gmaxp, h, zeros1, zeros2)


def _k4_body(acc0_r, acc1_r, den0_r, den1_r, sae0_r, sae1_r,
             deg0_r, deg1_r, asrc_r, adst_r, gmax_r, h_r, bias_r,
             out_r):
    gm = jnp.max(gmax_r[...])
    deg = deg0_r[...] + deg1_r[...]
    sae = sae0_r[...] + sae1_r[...]
    aloop = asrc_r[...] + adst_r[...] + sae / jnp.maximum(deg, 1.0)
    aloop = jnp.maximum(aloop, aloop * 0.2)
    el = jnp.exp(aloop - gm)
    den = den0_r[...] + den1_r[...] + el + 1e-16
    num = acc0_r[...] + acc1_r[...] + el * h_r[...]
    out = num / den + bias_r[...]
    out_r[...] = jnp.maximum(out, 0.0)


def _k4(acc, denomp, saep, degp, asrc, adst, gmaxp, h, bias):
    nb = NPAD // 8
    nblk = NPAD // nb

    def lo(i):
        return (i, 0)

    def hi(i):
        return (i + nblk, 0)

    den2 = denomp.reshape(NC * NPAD, 1)
    sae2 = saep.reshape(NC * NPAD, 1)
    deg2 = degp.reshape(NC * NPAD, 1)
    return pl.pallas_call(
        _k4_body,
        grid=(8,),
        in_specs=[
            pl.BlockSpec((nb, D), lo),
            pl.BlockSpec((nb, D), hi),
            pl.BlockSpec((nb, 1), lo),
            pl.BlockSpec((nb, 1), hi),
            pl.BlockSpec((nb, 1), lo),
            pl.BlockSpec((nb, 1), hi),
            pl.BlockSpec((nb, 1), lo),
            pl.BlockSpec((nb, 1), hi),
            pl.BlockSpec((nb, 1), lo),
            pl.BlockSpec((nb, 1), lo),
            pl.BlockSpec((8, 128), lambda i: (0, 0)),
            pl.BlockSpec((nb, D), lo),
            pl.BlockSpec((1, D), lambda i: (0, 0)),
        ],
        out_specs=pl.BlockSpec((nb, D), lo),
        out_shape=jax.ShapeDtypeStruct((N, D), jnp.float32),
    )(acc, acc, den2, den2, sae2, sae2, deg2, deg2,
      asrc.reshape(NPAD, 1), adst.reshape(NPAD, 1), gmaxp, h, bias)


@jax.jit
def kernel(x, edge_index, edge_attr, W, att_src, att_dst, lin_edge_W,
           att_edge, bias):
    src = edge_index[0]
    dst = edge_index[1]
    pad_nodes = (N + (jnp.arange(PAD_E, dtype=jnp.int32) % (NPAD - N)))
    comb = src | (dst << 16)
    combp = jnp.concatenate(
        [comb, pad_nodes | (pad_nodes << 16)]).reshape(NW, ECH, 128)
    ea2 = edge_attr.reshape(ER, 128)
    xp = jnp.concatenate([x, jnp.zeros((NPAD - N, D), jnp.float32)], axis=0)
    zeros1 = jnp.zeros((640,), jnp.float32)
    zeros2 = jnp.zeros((640, D), jnp.float32)
    ii = lax.broadcasted_iota(jnp.int32, (128, DE), 0)
    kk = lax.broadcasted_iota(jnp.int32, (128, DE), 1)
    emat = ((ii % DE) == kk).astype(jnp.float32)
    i8 = lax.broadcasted_iota(jnp.int32, (128, 8), 0)
    j8 = lax.broadcasted_iota(jnp.int32, (128, 8), 1)
    mask8 = ((i8 // DE) == j8).astype(jnp.float32)

    h, asrc3, adst3 = _k0a(xp, W, att_src.reshape(1, D),
                           att_dst.reshape(1, D))
    asrc = asrc3.reshape(NPAD)
    adst = adst3.reshape(NPAD)
    asum, degp = _k1a(combp, asrc, adst, zeros1)

    ae = _k0b(ea2, lin_edge_W, att_edge.reshape(1, D), emat, mask8)
    aef = jnp.concatenate(
        [ae.reshape(E), jnp.zeros((PAD_E,), jnp.float32)])

    alpha2, gmaxp = _k2(asum.reshape(EROWS, 128), aef.reshape(EROWS, 128))
    denomp, saep, acc = _k3(combp, alpha2, aef.reshape(EROWS, 128), gmaxp,
                            h, zeros1, zeros2)
    return _k4(acc, denomp, saep, degp, asrc, adst, gmaxp, h,
               bias.reshape(1, D))


# speedup vs baseline: 28.3227x; 1.0004x over previous
import jax
import jax.numpy as jnp
from jax import lax
from jax.experimental import pallas as pl
from jax.experimental.pallas import tpu as pltpu
from jax.experimental.pallas import tpu_sc as plsc

N = 10000
E = 320000
D = 128
DE = 16

NC = 2
NS = 16
NW = NC * NS
LANE = 16

NPAD = 10240
ECH = 79
ET = ECH * 128
EPAD = NW * ET
PAD_E = EPAD - E
EROWS = NW * ECH

ER = E // 8
EB = ER // 8
NB = NPAD // 8


def _k0a_body(x_r, w_r, asw_r, adw_r, h_r, asrc_r, adst_r):
    xb = x_r[...]
    h = lax.dot_general(xb, w_r[...], (((1,), (1,)), ((), ())),
                        preferred_element_type=jnp.float32)
    h_r[...] = h
    asrc_r[...] = jnp.sum(h * asw_r[...], axis=1)[None, None, :]
    adst_r[...] = jnp.sum(h * adw_r[...], axis=1)[None, None, :]


def _k0a(xp, W, att_src, att_dst):
    return pl.pallas_call(
        _k0a_body,
        grid=(8,),
        in_specs=[
            pl.BlockSpec((NB, D), lambda i: (i, 0)),
            pl.BlockSpec((D, D), lambda i: (0, 0)),
            pl.BlockSpec((1, D), lambda i: (0, 0)),
            pl.BlockSpec((1, D), lambda i: (0, 0)),
        ],
        out_specs=[
            pl.BlockSpec((NB, D), lambda i: (i, 0)),
            pl.BlockSpec((1, 1, NB), lambda i: (i, 0, 0)),
            pl.BlockSpec((1, 1, NB), lambda i: (i, 0, 0)),
        ],
        out_shape=[
            jax.ShapeDtypeStruct((NPAD, D), jnp.float32),
            jax.ShapeDtypeStruct((8, 1, NB), jnp.float32),
            jax.ShapeDtypeStruct((8, 1, NB), jnp.float32),
        ],
    )(xp, W, att_src, att_dst)


def _k0b_body(ea_r, lew_r, aew_r, emat_r, mask8_r, ae_r):
    v = lax.dot_general(aew_r[...], lew_r[...], (((1,), (0,)), ((), ())),
                        preferred_element_type=jnp.float32)
    vcol = lax.dot_general(emat_r[...], v, (((1,), (1,)), ((), ())),
                           preferred_element_type=jnp.float32)
    bmat = mask8_r[...] * vcol
    ae_r[...] = lax.dot_general(ea_r[...], bmat, (((1,), (0,)), ((), ())),
                                preferred_element_type=jnp.float32)


def _k0b(ea2, lin_edge_W, att_edge, emat, mask8):
    return pl.pallas_call(
        _k0b_body,
        grid=(8,),
        in_specs=[
            pl.BlockSpec((EB, 128), lambda i: (i, 0)),
            pl.BlockSpec((D, DE), lambda i: (0, 0)),
            pl.BlockSpec((1, D), lambda i: (0, 0)),
            pl.BlockSpec((128, DE), lambda i: (0, 0)),
            pl.BlockSpec((128, 8), lambda i: (0, 0)),
        ],
        out_specs=pl.BlockSpec((EB, 8), lambda i: (i, 0)),
        out_shape=jax.ShapeDtypeStruct((ER, 8), jnp.float32),
    )(ea2, lin_edge_W, att_edge, emat, mask8)


def _k1a_body(combp, asrc, adst, zeros1,
              asum_o, degp_o,
              comb_v, dst_v, asum_v, asrc_v, adst_v, ones_v,
              deg_sh):
    cid = lax.axis_index("c")
    sid = lax.axis_index("s")
    wid = sid * NC + cid

    pltpu.sync_copy(combp.at[wid], comb_v)
    pltpu.sync_copy(asrc, asrc_v)
    pltpu.sync_copy(adst, adst_v)

    for g in range(8):
        ones_v[pl.ds(g * LANE, LANE)] = jnp.full((LANE,), 1.0, jnp.float32)

    pltpu.sync_copy(zeros1, deg_sh.at[pl.ds(sid * 640, 640)])
    plsc.subcore_barrier()

    def chunk(j, c):
        for g in range(8):
            off = pl.ds(g * LANE, LANE)
            c16 = comb_v[j, off]
            s16 = c16 & 0xFFFF
            d16 = lax.shift_right_logical(c16, 16)
            dst_v[j, off] = d16
            asum_v[j, off] = plsc.load_gather(asrc_v, [s16]) \
                + plsc.load_gather(adst_v, [d16])
        pltpu.sync_copy(ones_v, deg_sh.at[dst_v.at[j]], add=True)
        return c

    lax.fori_loop(0, ECH, chunk, 0)

    pltpu.sync_copy(asum_v, asum_o.at[wid])
    plsc.subcore_barrier()
    base = cid * NPAD + sid * 640
    pltpu.sync_copy(deg_sh.at[pl.ds(sid * 640, 640)],
                    degp_o.at[pl.ds(base, 640)])


def _k1a(combp, asrc, adst, zeros1):
    mesh = plsc.VectorSubcoreMesh(core_axis_name="c", subcore_axis_name="s",
                                  num_cores=NC, num_subcores=NS)
    f = pl.kernel(
        _k1a_body,
        out_type=[
            jax.ShapeDtypeStruct((NW, ECH, 128), jnp.float32),
            jax.ShapeDtypeStruct((NC * NPAD,), jnp.float32),
        ],
        mesh=mesh,
        compiler_params=pltpu.CompilerParams(needs_layout_passes=False),
        scratch_types=[
            pltpu.VMEM((ECH, 128), jnp.int32),
            pltpu.VMEM((ECH, 128), jnp.int32),
            pltpu.VMEM((ECH, 128), jnp.float32),
            pltpu.VMEM((NPAD,), jnp.float32),
            pltpu.VMEM((NPAD,), jnp.float32),
            pltpu.VMEM((128,), jnp.float32),
            pltpu.VMEM_SHARED((NPAD,), jnp.float32),
        ],
    )
    return f(combp, asrc, adst, zeros1)


def _k2_body(asum_r, ae_r, alpha_o, gmax_o):
    al = asum_r[...] + ae_r[...]
    al = jnp.maximum(al, al * 0.2)
    alpha_o[...] = al
    m = jnp.max(al)
    i = pl.program_id(0)

    @pl.when(i == 0)
    def _():
        gmax_o[...] = jnp.full((8, 128), m, jnp.float32)

    @pl.when(i > 0)
    def _():
        gmax_o[...] = jnp.maximum(gmax_o[...], m)


def _k2(asum, ae2):
    kb = EROWS // 4
    return pl.pallas_call(
        _k2_body,
        grid=(4,),
        in_specs=[
            pl.BlockSpec((kb, 128), lambda i: (i, 0)),
            pl.BlockSpec((kb, 128), lambda i: (i, 0)),
        ],
        out_specs=[
            pl.BlockSpec((kb, 128), lambda i: (i, 0)),
            pl.BlockSpec((8, 128), lambda i: (0, 0)),
        ],
        out_shape=[
            jax.ShapeDtypeStruct((EROWS, 128), jnp.float32),
            jax.ShapeDtypeStruct((8, 128), jnp.float32),
        ],
    )(asum, ae2)


def _k3_body(combp, alpha2, aerows, gmaxp, h, zeros1, zeros2,
             denomp_o, saep_o, acc_o,
             comb_v, src_a, src_b, dst_a, dst_b, exp_a, exp_b,
             ae_a, ae_b, rows_a, rows_b, gmax_v,
             denom_sh, sae_sh, acc_sh, sem_a, sem_b):
    cid = lax.axis_index("c")
    sid = lax.axis_index("s")
    wid = sid * NC + cid

    pltpu.sync_copy(combp.at[wid], comb_v)
    pltpu.sync_copy(gmaxp.at[0], gmax_v)

    pltpu.sync_copy(zeros1, denom_sh.at[pl.ds(sid * 640, 640)])
    pltpu.sync_copy(zeros1, sae_sh.at[pl.ds(sid * 640, 640)])
    pltpu.sync_copy(zeros2, acc_sh.at[pl.ds(sid * 640, 640)])
    plsc.subcore_barrier()

    gsp = gmax_v[pl.ds(0, LANE)]
    arow0 = wid * ECH

    def prep(j, src_r, dst_r, exp_r, ae_r, rows, sem):
        for g in range(8):
            off = pl.ds(g * LANE, LANE)
            c16 = comb_v[j, off]
            src_r[off] = c16 & 0xFFFF
            dst_r[off] = lax.shift_right_logical(c16, 16)
        pltpu.async_copy(alpha2.at[arow0 + j], exp_r, sem)
        pltpu.async_copy(aerows.at[arow0 + j], ae_r, sem)
        pltpu.async_copy(h.at[src_r], rows, sem)

    def process(j, src_r, dst_r, exp_r, ae_r, rows, sem):
        pltpu.make_async_copy(alpha2.at[arow0 + j], exp_r, sem).wait()
        pltpu.make_async_copy(aerows.at[arow0 + j], ae_r, sem).wait()
        pltpu.make_async_copy(h.at[src_r], rows, sem).wait()
        for g in range(8):
            off = pl.ds(g * LANE, LANE)
            exp_r[off] = jnp.exp(exp_r[off] - gsp)
        pltpu.sync_copy(exp_r, denom_sh.at[dst_r], add=True)
        pltpu.sync_copy(ae_r, sae_sh.at[dst_r], add=True)

        def scale(r, c):
            spl = plsc.load_gather(exp_r, [jnp.full((LANE,), r, jnp.int32)])
            for g in range(8):
                off = pl.ds(g * LANE, LANE)
                rows[r, off] = rows[r, off] * spl
            return c
        lax.fori_loop(0, 128, scale, 0)
        pltpu.sync_copy(rows, acc_sh.at[dst_r], add=True)

    prep(0, src_a, dst_a, exp_a, ae_a, rows_a, sem_a)

    def pair(j2, carry):
        ja = 2 * j2
        jb = ja + 1
        prep(jb, src_b, dst_b, exp_b, ae_b, rows_b, sem_b)
        process(ja, src_a, dst_a, exp_a, ae_a, rows_a, sem_a)
        prep(ja + 2, src_a, dst_a, exp_a, ae_a, rows_a, sem_a)
        process(jb, src_b, dst_b, exp_b, ae_b, rows_b, sem_b)
        return carry

    lax.fori_loop(0, (ECH - 1) // 2, pair, 0)
    process(ECH - 1, src_a, dst_a, exp_a, ae_a, rows_a, sem_a)

    plsc.subcore_barrier()
    base = cid * NPAD + sid * 640
    pltpu.sync_copy(denom_sh.at[pl.ds(sid * 640, 640)],
                    denomp_o.at[pl.ds(base, 640)])
    pltpu.sync_copy(sae_sh.at[pl.ds(sid * 640, 640)],
                    saep_o.at[pl.ds(base, 640)])
    pltpu.sync_copy(acc_sh.at[pl.ds(sid * 640, 640)],
                    acc_o.at[pl.ds(base, 640)])


def _k3(combp, alpha2, aerows, gmaxp, h, zeros1, zeros2):
    mesh = plsc.VectorSubcoreMesh(core_axis_name="c", subcore_axis_name="s",
                                  num_cores=NC, num_subcores=NS)
    f = pl.kernel(
        _k3_body,
        out_type=[
            jax.ShapeDtypeStruct((NC * NPAD,), jnp.float32),
            jax.ShapeDtypeStruct((NC * NPAD,), jnp.float32),
            jax.ShapeDtypeStruct((NC * NPAD, D), jnp.float32),
        ],
        mesh=mesh,
        compiler_params=pltpu.CompilerParams(needs_layout_passes=False),
        scratch_types=[
            pltpu.VMEM((ECH, 128), jnp.int32),
            pltpu.VMEM((128,), jnp.int32),
            pltpu.VMEM((128,), jnp.int32),
            pltpu.VMEM((128,), jnp.int32),
            pltpu.VMEM((128,), jnp.int32),
            pltpu.VMEM((128,), jnp.float32),
            pltpu.VMEM((128,), jnp.float32),
            pltpu.VMEM((128,), jnp.float32),
            pltpu.VMEM((128,), jnp.float32),
            pltpu.VMEM((128, D), jnp.float32),
            pltpu.VMEM((128, D), jnp.float32),
            pltpu.VMEM((128,), jnp.float32),
            pltpu.VMEM_SHARED((NPAD,), jnp.float32),
            pltpu.VMEM_SHARED((NPAD,), jnp.float32),
            pltpu.VMEM_SHARED((NPAD, D), jnp.float32),
            pltpu.SemaphoreType.DMA,
            pltpu.SemaphoreType.DMA,
        ],
    )
    return f(combp, alpha2, aerows, gmaxp, h, zeros1, zeros2)


def _k4_body(acc0_r, acc1_r, den0_r, den1_r, sae0_r, sae1_r,
             deg0_r, deg1_r, asrc_r, adst_r, gmax_r, h_r, bias_r,
             out_r):
    gm = jnp.max(gmax_r[...])
    deg = deg0_r[...] + deg1_r[...]
    sae = sae0_r[...] + sae1_r[...]
    aloop = asrc_r[...] + adst_r[...] + sae / jnp.maximum(deg, 1.0)
    aloop = jnp.maximum(aloop, aloop * 0.2)
    el = jnp.exp(aloop - gm)
    den = den0_r[...] + den1_r[...] + el + 1e-16
    num = acc0_r[...] + acc1_r[...] + el * h_r[...]
    out = num / den + bias_r[...]
    out_r[...] = jnp.maximum(out, 0.0)


def _k4(acc, denomp, saep, degp, asrc, adst, gmaxp, h, bias):
    nb = NPAD // 8
    nblk = NPAD // nb

    def lo(i):
        return (i, 0)

    def hi(i):
        return (i + nblk, 0)

    den2 = denomp.reshape(NC * NPAD, 1)
    sae2 = saep.reshape(NC * NPAD, 1)
    deg2 = degp.reshape(NC * NPAD, 1)
    return pl.pallas_call(
        _k4_body,
        grid=(8,),
        in_specs=[
            pl.BlockSpec((nb, D), lo),
            pl.BlockSpec((nb, D), hi),
            pl.BlockSpec((nb, 1), lo),
            pl.BlockSpec((nb, 1), hi),
            pl.BlockSpec((nb, 1), lo),
            pl.BlockSpec((nb, 1), hi),
            pl.BlockSpec((nb, 1), lo),
            pl.BlockSpec((nb, 1), hi),
            pl.BlockSpec((nb, 1), lo),
            pl.BlockSpec((nb, 1), lo),
            pl.BlockSpec((8, 128), lambda i: (0, 0)),
            pl.BlockSpec((nb, D), lo),
            pl.BlockSpec((1, D), lambda i: (0, 0)),
        ],
        out_specs=pl.BlockSpec((nb, D), lo),
        out_shape=jax.ShapeDtypeStruct((N, D), jnp.float32),
    )(acc, acc, den2, den2, sae2, sae2, deg2, deg2,
      asrc.reshape(NPAD, 1), adst.reshape(NPAD, 1), gmaxp, h, bias)


@jax.jit
def kernel(x, edge_index, edge_attr, W, att_src, att_dst, lin_edge_W,
           att_edge, bias):
    src = edge_index[0]
    dst = edge_index[1]
    pad_nodes = (N + (jnp.arange(PAD_E, dtype=jnp.int32) % (NPAD - N)))
    comb = src | (dst << 16)
    combp = jnp.concatenate(
        [comb, pad_nodes | (pad_nodes << 16)]).reshape(NW, ECH, 128)
    ea2 = edge_attr.reshape(ER, 128)
    xp = jnp.concatenate([x, jnp.zeros((NPAD - N, D), jnp.float32)], axis=0)
    zeros1 = jnp.zeros((640,), jnp.float32)
    zeros2 = jnp.zeros((640, D), jnp.float32)
    ii = lax.broadcasted_iota(jnp.int32, (128, DE), 0)
    kk = lax.broadcasted_iota(jnp.int32, (128, DE), 1)
    emat = ((ii % DE) == kk).astype(jnp.float32)
    i8 = lax.broadcasted_iota(jnp.int32, (128, 8), 0)
    j8 = lax.broadcasted_iota(jnp.int32, (128, 8), 1)
    mask8 = ((i8 // DE) == j8).astype(jnp.float32)

    h, asrc3, adst3 = _k0a(xp, W, att_src.reshape(1, D),
                           att_dst.reshape(1, D))
    asrc = asrc3.reshape(NPAD)
    adst = adst3.reshape(NPAD)
    asum, degp = _k1a(combp, asrc, adst, zeros1)

    ae = _k0b(ea2, lin_edge_W, att_edge.reshape(1, D), emat, mask8)
    aef = jnp.concatenate(
        [ae.reshape(E), jnp.zeros((PAD_E,), jnp.float32)])

    alpha2, gmaxp = _k2(asum.reshape(EROWS, 128), aef.reshape(EROWS, 128))
    denomp, saep, acc = _k3(combp, alpha2, aef.reshape(EROWS, 128), gmaxp,
                            h, zeros1, zeros2)
    return _k4(acc, denomp, saep, degp, asrc, adst, gmaxp, h,
               bias.reshape(1, D))
